# baseline (speedup 1.0000x reference)
import functools

import jax
import jax.numpy as jnp
from jax import lax
from jax.experimental import pallas as pl
from jax.experimental.pallas import tpu as pltpu
from jax.experimental.pallas import tpu_sc as plsc

B = 16384
D = 128
D2 = 2 * D
NC = 2
NS = 16
NW = NC * NS
CH = 128


@functools.cache
def _make_gather(nb, cbase0):
    bpw = nb // NW
    nch = bpw // CH
    mesh = plsc.VectorSubcoreMesh(core_axis_name="c", subcore_axis_name="s")

    @functools.partial(
        pl.kernel,
        mesh=mesh,
        out_type=jax.ShapeDtypeStruct((nb, D2), jnp.float32),
        scratch_types=[
            pltpu.VMEM((nch, CH), jnp.int32),
            pltpu.VMEM((nch, CH), jnp.int32),
            pltpu.VMEM((CH, D), jnp.float32),
            pltpu.VMEM((CH, D), jnp.float32),
            pltpu.VMEM((CH, D), jnp.float32),
            pltpu.VMEM((CH, D), jnp.float32),
            pltpu.SemaphoreType.DMA,
            pltpu.SemaphoreType.DMA,
        ],
    )
    def gather_k(uids2, bids2, utab, btab, x_out,
                 uidx_v, bidx_v, ur0, ur1, br0, br1, sg, sw):
        wid = lax.axis_index("s") * NC + lax.axis_index("c")
        cbase = wid * nch
        pltpu.sync_copy(uids2.at[pl.ds(cbase0 + cbase, nch)], uidx_v)
        pltpu.sync_copy(bids2.at[pl.ds(cbase0 + cbase, nch)], bidx_v)
        ubufs, bbufs = (ur0, ur1), (br0, br1)
        gathers = {}
        writes = {}

        def fire_gather(c):
            gu = pltpu.async_copy(utab.at[uidx_v.at[c]], ubufs[c % 2], sg)
            gb = pltpu.async_copy(btab.at[bidx_v.at[c]], bbufs[c % 2], sg)
            gathers[c] = (gu, gb)

        fire_gather(0)
        for c in range(nch):
            if c + 1 < nch:
                if c - 1 >= 0:
                    for w in writes.pop(c - 1):
                        w.wait()
                fire_gather(c + 1)
            gu, gb = gathers.pop(c)
            gu.wait()
            gb.wait()
            off = (cbase + c) * CH
            wu = pltpu.async_copy(
                ubufs[c % 2], x_out.at[pl.ds(off, CH), pl.ds(0, D)], sw)
            wb = pltpu.async_copy(
                bbufs[c % 2], x_out.at[pl.ds(off, CH), pl.ds(D, D)], sw)
            writes[c] = (wu, wb)
        for c in sorted(writes):
            for w in writes[c]:
                w.wait()

    return gather_k


CHUNK = 1024


def _mlp_body(x_ref, w0_ref, b0_ref, w1_ref, b1_ref,
              w2_ref, b2_ref, w3_ref, b3_ref, out_ref):
    bf = jnp.bfloat16
    h = jnp.dot(x_ref[...].astype(bf), w0_ref[...],
                preferred_element_type=jnp.float32)
    h = jnp.maximum(h + b0_ref[...][None, :], 0.0)
    h = jnp.dot(h.astype(bf), w1_ref[...], preferred_element_type=jnp.float32)
    h = jnp.maximum(h + b1_ref[...][None, :], 0.0)
    h = jnp.dot(h.astype(bf), w2_ref[...], preferred_element_type=jnp.float32)
    h = jnp.maximum(h + b2_ref[...][None, :], 0.0)
    r = jax.lax.dot_general(w3_ref[...], h.astype(bf),
                            dimension_numbers=(((1,), (1,)), ((), ())),
                            preferred_element_type=jnp.float32)
    out_ref[...] = r[0] + b3_ref[0, 0]


def _mlp(nb, x, W0, b0, W1, b1, W2, b2, w3, b3):
    return pl.pallas_call(
        _mlp_body,
        grid=(nb // CHUNK,),
        in_specs=[
            pl.BlockSpec((CHUNK, D2), lambda i: (i, 0)),
            pl.BlockSpec((D2, 512), lambda i: (0, 0)),
            pl.BlockSpec((512,), lambda i: (0,)),
            pl.BlockSpec((512, 256), lambda i: (0, 0)),
            pl.BlockSpec((256,), lambda i: (0,)),
            pl.BlockSpec((256, 128), lambda i: (0, 0)),
            pl.BlockSpec((128,), lambda i: (0,)),
            pl.BlockSpec((1, 128), lambda i: (0, 0)),
            pl.BlockSpec((1, 1), lambda i: (0, 0)),
        ],
        out_specs=pl.BlockSpec((CHUNK,), lambda i: (i,)),
        out_shape=jax.ShapeDtypeStruct((nb,), jnp.float32),
        compiler_params=pltpu.CompilerParams(
            dimension_semantics=("parallel",),
        ),
    )(x, W0, b0, W1, b1, W2, b2, w3, b3)


NSPLIT = 4


def kernel(user_ids, book_ids, user_table, book_table,
           W0, b0, W1, b1, W2, b2, W3, b3):
    bf = jnp.bfloat16
    w = (W0.astype(bf), b0, W1.astype(bf), b1, W2.astype(bf), b2,
         W3.reshape(1, 128).astype(bf), b3[None, :])
    h = B // NSPLIT
    uids2 = user_ids.astype(jnp.int32).reshape(-1, CH)
    bids2 = book_ids.astype(jnp.int32).reshape(-1, CH)
    rph = h // CH
    outs = []
    for s in range(NSPLIT):
        x = _make_gather(h, s * rph)(uids2, bids2, user_table, book_table)
        outs.append(_mlp(h, x, *w))
    return jnp.concatenate(outs, axis=0)


# speedup vs baseline: 1.0702x; 1.0702x over previous
import functools

import jax
import jax.numpy as jnp
from jax import lax
from jax.experimental import pallas as pl
from jax.experimental.pallas import tpu as pltpu
from jax.experimental.pallas import tpu_sc as plsc

B = 16384
D = 128
D2 = 2 * D
NC = 2
NS = 16
NW = NC * NS
CH = 128


@functools.cache
def _make_gather(nb, cbase0):
    bpw = nb // NW
    nch = bpw // CH
    mesh = plsc.VectorSubcoreMesh(core_axis_name="c", subcore_axis_name="s")

    @functools.partial(
        pl.kernel,
        mesh=mesh,
        out_type=jax.ShapeDtypeStruct((nb, D2), jnp.float32),
        scratch_types=[
            pltpu.VMEM((nch, CH), jnp.int32),
            pltpu.VMEM((nch, CH), jnp.int32),
            pltpu.VMEM((CH, D), jnp.float32),
            pltpu.VMEM((CH, D), jnp.float32),
            pltpu.VMEM((CH, D), jnp.float32),
            pltpu.VMEM((CH, D), jnp.float32),
            pltpu.SemaphoreType.DMA,
            pltpu.SemaphoreType.DMA,
        ],
    )
    def gather_k(uids2, bids2, utab, btab, x_out,
                 uidx_v, bidx_v, ur0, ur1, br0, br1, sg, sw):
        wid = lax.axis_index("s") * NC + lax.axis_index("c")
        cbase = wid * nch
        pltpu.sync_copy(uids2.at[pl.ds(cbase0 + cbase, nch)], uidx_v)
        pltpu.sync_copy(bids2.at[pl.ds(cbase0 + cbase, nch)], bidx_v)
        ubufs, bbufs = (ur0, ur1), (br0, br1)
        gathers = {}
        writes = {}

        def fire_gather(c):
            gu = pltpu.async_copy(utab.at[uidx_v.at[c]], ubufs[c % 2], sg)
            gb = pltpu.async_copy(btab.at[bidx_v.at[c]], bbufs[c % 2], sg)
            gathers[c] = (gu, gb)

        fire_gather(0)
        for c in range(nch):
            if c + 1 < nch:
                if c - 1 >= 0:
                    for w in writes.pop(c - 1):
                        w.wait()
                fire_gather(c + 1)
            gu, gb = gathers.pop(c)
            gu.wait()
            gb.wait()
            off = (cbase + c) * CH
            wu = pltpu.async_copy(
                ubufs[c % 2], x_out.at[pl.ds(off, CH), pl.ds(0, D)], sw)
            wb = pltpu.async_copy(
                bbufs[c % 2], x_out.at[pl.ds(off, CH), pl.ds(D, D)], sw)
            writes[c] = (wu, wb)
        for c in sorted(writes):
            for w in writes[c]:
                w.wait()

    return gather_k


CHUNK = 1024


def _mlp_body(x_ref, w0_ref, b0_ref, w1_ref, b1_ref,
              w2_ref, b2_ref, w3_ref, b3_ref, prev_ref, out_ref):
    bf = jnp.bfloat16
    h = jnp.dot(x_ref[...].astype(bf), w0_ref[...],
                preferred_element_type=jnp.float32)
    h = jnp.maximum(h + b0_ref[...][None, :], 0.0)
    h = jnp.dot(h.astype(bf), w1_ref[...], preferred_element_type=jnp.float32)
    h = jnp.maximum(h + b1_ref[...][None, :], 0.0)
    h = jnp.dot(h.astype(bf), w2_ref[...], preferred_element_type=jnp.float32)
    h = jnp.maximum(h + b2_ref[...][None, :], 0.0)
    r = jax.lax.dot_general(w3_ref[...], h.astype(bf),
                            dimension_numbers=(((1,), (1,)), ((), ())),
                            preferred_element_type=jnp.float32)
    out_ref[...] = r[0] + b3_ref[0, 0]


def _mlp(nb, bofs, x, W0, b0, W1, b1, W2, b2, w3, b3, prev):
    return pl.pallas_call(
        _mlp_body,
        grid=(nb // CHUNK,),
        in_specs=[
            pl.BlockSpec((CHUNK, D2), lambda i: (i, 0)),
            pl.BlockSpec((D2, 512), lambda i: (0, 0)),
            pl.BlockSpec((512,), lambda i: (0,)),
            pl.BlockSpec((512, 256), lambda i: (0, 0)),
            pl.BlockSpec((256,), lambda i: (0,)),
            pl.BlockSpec((256, 128), lambda i: (0, 0)),
            pl.BlockSpec((128,), lambda i: (0,)),
            pl.BlockSpec((1, 128), lambda i: (0, 0)),
            pl.BlockSpec((1, 1), lambda i: (0, 0)),
            pl.BlockSpec(memory_space=pl.ANY),
        ],
        out_specs=pl.BlockSpec((CHUNK,), lambda i: (i + bofs,)),
        out_shape=jax.ShapeDtypeStruct((B,), jnp.float32),
        input_output_aliases={9: 0},
        compiler_params=pltpu.CompilerParams(
            dimension_semantics=("parallel",),
        ),
    )(x, W0, b0, W1, b1, W2, b2, w3, b3, prev)


NSPLIT = 2


def kernel(user_ids, book_ids, user_table, book_table,
           W0, b0, W1, b1, W2, b2, W3, b3):
    bf = jnp.bfloat16
    w = (W0.astype(bf), b0, W1.astype(bf), b1, W2.astype(bf), b2,
         W3.reshape(1, 128).astype(bf), b3[None, :])
    h = B // NSPLIT
    uids2 = user_ids.astype(jnp.int32).reshape(-1, CH)
    bids2 = book_ids.astype(jnp.int32).reshape(-1, CH)
    rph = h // CH
    out = jnp.zeros((B,), jnp.float32)
    for s in range(NSPLIT):
        x = _make_gather(h, s * rph)(uids2, bids2, user_table, book_table)
        out = _mlp(h, s * (h // CHUNK), x, *w, out)
    return out
